# Initial kernel scaffold; baseline (speedup 1.0000x reference)
import jax
import jax.numpy as jnp
from jax.experimental import pallas as pl


def kernel(emb0, emb1, emb2, emb3, lw0, lw1, lw2, lw3, lb0, lb1, lb2, lb3, ww0, ww1, wb0, wb1, pre_adj0, pre_adj1, idx):
    raise NotImplementedError("write your pallas kernel here")


# speedup vs baseline: 1.8300x; 1.8300x over previous
import jax
import jax.numpy as jnp
from jax.experimental import pallas as pl
from jax.experimental.pallas import tpu as pltpu

N1 = 2048
DIM = 64
K = 20
ALPHA = 3.0
NN = 2 * N1
TILE = 128
NT = N1 // TILE


def _dot_t(a, b):
    return jax.lax.dot_general(
        a, b, (((1,), (1,)), ((), ())),
        precision=jax.lax.Precision.HIGHEST,
        preferred_element_type=jnp.float32)


def _fused(emb_ref, lw_ref, lb_ref, ww_ref, wb_ref, pre_ref, out_ref,
           nv1_ref, nv2_ref):
    r = pl.program_id(0)
    t = pl.program_id(1)

    def make_nv(rr):
        for j in range(2):
            i1 = 2 * rr + j
            i2 = 2 * j + rr
            nv1_ref[j] = jnp.tanh(
                ALPHA * (_dot_t(emb_ref[i1], lw_ref[i1]) + lb_ref[i1]))
            nv2_ref[j] = jnp.tanh(
                ALPHA * (_dot_t(emb_ref[i2], lw_ref[i1]) + lb_ref[i1]))

    @pl.when((t == 0) & (r == 0))
    def _():
        make_nv(0)

    @pl.when((t == 0) & (r == 1))
    def _():
        make_nv(1)

    accw = _dot_t(pre_ref[0], ww_ref[0]) + wb_ref[0]
    row0 = nv1_ref[0, pl.ds(t * TILE, TILE), :]
    row1 = nv1_ref[1, pl.ds(t * TILE, TILE), :]
    log0 = _dot_t(row0, nv2_ref[0]) + accw
    log1 = _dot_t(row1, nv2_ref[1]) + accw
    logits = jnp.concatenate([log0, log1], axis=1)

    adj = jnp.maximum(jnp.tanh(ALPHA * logits), 0.0)

    bits = jax.lax.bitcast_convert_type(adj, jnp.int32)
    lo = jnp.full((TILE, 1), -1, jnp.int32)
    hi = jnp.full((TILE, 1), 0x3F800000, jnp.int32)
    for _ in range(31):
        mid = (lo + hi) >> 1
        cnt = jnp.sum((bits > mid).astype(jnp.int32), axis=1, keepdims=True)
        ge = cnt >= K
        lo = jnp.where(ge, mid, lo)
        hi = jnp.where(ge, hi, mid)
    thr = hi
    n_gt = jnp.sum((bits > thr).astype(jnp.int32), axis=1, keepdims=True)
    m_tie = K - n_gt
    tie = bits == thr
    iota = jax.lax.broadcasted_iota(jnp.int32, (TILE, NN), 1)
    ilo = jnp.full((TILE, 1), -1, jnp.int32)
    ihi = jnp.full((TILE, 1), NN - 1, jnp.int32)
    for _ in range(12):
        mid = (ilo + ihi) >> 1
        c = jnp.sum((tie & (iota <= mid)).astype(jnp.int32), axis=1,
                    keepdims=True)
        ok = c >= m_tie
        ihi = jnp.where(ok, mid, ihi)
        ilo = jnp.where(ok, ilo, mid)
    mask = (bits > thr) | (tie & (iota <= ihi))
    out_ref[...] = jnp.where(mask, adj, 0.0)


def kernel(emb0, emb1, emb2, emb3, lw0, lw1, lw2, lw3, lb0, lb1, lb2, lb3,
           ww0, ww1, wb0, wb1, pre_adj0, pre_adj1, idx):
    emb = jnp.stack([emb0, emb1, emb2, emb3])
    lw = jnp.stack([lw0, lw1, lw2, lw3])
    lb = jnp.stack([lb0, lb1, lb2, lb3])[:, None, :]
    ww = jnp.stack([ww0, ww1])
    wb = jnp.stack([wb0, wb1])[:, None, :]
    pre = jnp.stack([pre_adj0, pre_adj1])

    return pl.pallas_call(
        _fused,
        grid=(2, NT),
        in_specs=[
            pl.BlockSpec((4, N1, DIM), lambda r, t: (0, 0, 0)),
            pl.BlockSpec((4, DIM, DIM), lambda r, t: (0, 0, 0)),
            pl.BlockSpec((4, 1, DIM), lambda r, t: (0, 0, 0)),
            pl.BlockSpec((1, N1, N1), lambda r, t: (r, 0, 0)),
            pl.BlockSpec((1, 1, N1), lambda r, t: (r, 0, 0)),
            pl.BlockSpec((1, TILE, N1), lambda r, t: (r, t, 0)),
        ],
        out_specs=pl.BlockSpec((TILE, NN), lambda r, t: (r * NT + t, 0)),
        out_shape=jax.ShapeDtypeStruct((NN, NN), jnp.float32),
        scratch_shapes=[
            pltpu.VMEM((2, N1, DIM), jnp.float32),
            pltpu.VMEM((2, N1, DIM), jnp.float32),
        ],
    )(emb, lw, lb, ww, wb, pre)
